# baseline (speedup 1.0000x reference)
import functools

import jax
import jax.numpy as jnp
from jax import lax
from jax.experimental import pallas as pl
from jax.experimental.pallas import tpu as pltpu
from jax.experimental.pallas import tpu_sc as plsc

LANES = 16
NBUF = 3


def _make_sc_kernel(B, S, D, b_lo):
    nb = B - b_lo
    info = plsc.get_sparse_core_info()
    NC, NS = info.num_cores, info.num_subcores
    NW = NC * NS
    s_per_w = S // NW
    CH = 16 if nb == 1 else 8
    n_ch = s_per_w // CH
    n_col = D // LANES

    mesh = plsc.VectorSubcoreMesh(core_axis_name="c", subcore_axis_name="s")

    scratch = (
        [pltpu.VMEM((nb, CH, D), jnp.float32) for _ in range(NBUF)]
        + [pltpu.VMEM((CH, D), jnp.float32) for _ in range(NBUF)]
        + [pltpu.SemaphoreType.DMA for _ in range(2 * NBUF)]
    )

    @functools.partial(
        pl.kernel,
        mesh=mesh,
        out_type=jax.ShapeDtypeStruct((nb, S, D), jnp.float32),
        scratch_types=scratch,
        compiler_params=pltpu.CompilerParams(use_tc_tiling_on_sc=True),
    )
    def k(xf, pe, out, xb0, xb1, xb2, pb0, pb1, pb2,
          ls0, ls1, ls2, ss0, ss1, ss2):
        xbs = (xb0, xb1, xb2)
        pbs = (pb0, pb1, pb2)
        lss = (ls0, ls1, ls2)
        sss = (ss0, ss1, ss2)

        wid = lax.axis_index("s") * NC + lax.axis_index("c")
        s_base = wid * s_per_w

        def start_loads(c):
            p = c % NBUF
            s0 = s_base + c * CH
            return [
                pltpu.async_copy(pe.at[pl.ds(s0, CH), :], pbs[p], lss[p]),
                pltpu.async_copy(
                    xf.at[pl.ds(b_lo, nb), pl.ds(s0, CH), :], xbs[p], lss[p]),
            ]

        def start_stores(c):
            p = c % NBUF
            s0 = s_base + c * CH
            return [
                pltpu.async_copy(xbs[p], out.at[:, pl.ds(s0, CH), :], sss[p]),
            ]

        def compute(c):
            p = c % NBUF
            xb, pb = xbs[p], pbs[p]

            def body(r, carry):
                for g in range(n_col):
                    col = g * LANES
                    vec = pb[r, pl.ds(col, LANES)]
                    for b in range(nb):
                        plsc.addupdate(xb.at[b, r, pl.ds(col, LANES)], vec)
                return carry

            lax.fori_loop(0, CH, body, 0)

        loads = {c: start_loads(c) for c in range(min(NBUF, n_ch))}
        stores = {}
        for c in range(n_ch):
            if c >= NBUF - 1:
                for h in stores.pop(c - (NBUF - 1)):
                    h.wait()
                if c + 1 < n_ch:
                    loads[c + 1] = start_loads(c + 1)
            for h in loads.pop(c):
                h.wait()
            compute(c)
            stores[c] = start_stores(c)
        for hs in stores.values():
            for h in hs:
                h.wait()

    return k


def _tc_add_body(x_ref, pe_ref, o_ref):
    o_ref[...] = x_ref[...] + pe_ref[...]


def _tc_kernel(x, pe, nb_tc):
    B, S, D = x.shape
    bs = 256
    return pl.pallas_call(
        _tc_add_body,
        grid=(S // bs, nb_tc),
        in_specs=[
            pl.BlockSpec((1, bs, D), lambda s, b: (b, s, 0)),
            pl.BlockSpec((bs, D), lambda s, b: (s, 0)),
        ],
        out_specs=pl.BlockSpec((1, bs, D), lambda s, b: (b, s, 0)),
        out_shape=jax.ShapeDtypeStruct((nb_tc, S, D), x.dtype),
    )(x, pe)


def kernel(x, pe_weight):
    B, S, D = x.shape
    pe = pe_weight[:S]
    b_lo = B - 1
    sc_out = _make_sc_kernel(B, S, D, b_lo)(x, pe)
    tc_out = _tc_kernel(x, pe, b_lo)
    return jnp.concatenate([tc_out, sc_out], axis=0)


# speedup vs baseline: 1.2383x; 1.2383x over previous
import functools

import jax
import jax.numpy as jnp
from jax import lax
from jax.experimental import pallas as pl
from jax.experimental.pallas import tpu as pltpu
from jax.experimental.pallas import tpu_sc as plsc

LANES = 16
NBUF = 3
CH = 8


def _make_sc_kernel(B, S, D):
    info = plsc.get_sparse_core_info()
    NC, NS = info.num_cores, info.num_subcores
    NW = NC * NS
    s_per_w = S // NW
    n_ch = s_per_w // CH
    n_col = D // LANES

    mesh = plsc.VectorSubcoreMesh(core_axis_name="c", subcore_axis_name="s")

    scratch = (
        [pltpu.VMEM((B, CH, D), jnp.float32) for _ in range(NBUF)]
        + [pltpu.VMEM((CH, D), jnp.float32) for _ in range(NBUF)]
        + [pltpu.SemaphoreType.DMA for _ in range(2 * NBUF)]
    )

    @functools.partial(
        pl.kernel,
        mesh=mesh,
        out_type=jax.ShapeDtypeStruct((B, S, D), jnp.float32),
        scratch_types=scratch,
        compiler_params=pltpu.CompilerParams(use_tc_tiling_on_sc=True),
    )
    def k(xf, pe, out, *refs):
        xbs = refs[:NBUF]
        pbs = refs[NBUF:2 * NBUF]
        lss = refs[2 * NBUF:3 * NBUF]
        sss = refs[3 * NBUF:4 * NBUF]

        wid = lax.axis_index("s") * NC + lax.axis_index("c")
        s_base = wid * s_per_w

        def start_loads(c):
            p = c % NBUF
            s0 = s_base + c * CH
            return [
                pltpu.async_copy(pe.at[pl.ds(s0, CH), :], pbs[p], lss[p]),
                pltpu.async_copy(xf.at[:, pl.ds(s0, CH), :], xbs[p], lss[p]),
            ]

        def start_stores(c):
            p = c % NBUF
            s0 = s_base + c * CH
            return [
                pltpu.async_copy(xbs[p], out.at[:, pl.ds(s0, CH), :], sss[p]),
            ]

        def compute(c):
            p = c % NBUF
            xb, pb = xbs[p], pbs[p]

            def body(r, carry):
                for g in range(n_col):
                    col = g * LANES
                    vec = pb[r, pl.ds(col, LANES)]
                    for b in range(B):
                        plsc.addupdate(xb.at[b, r, pl.ds(col, LANES)], vec)
                return carry

            lax.fori_loop(0, CH, body, 0)

        loads = {c: start_loads(c) for c in range(min(NBUF, n_ch))}
        stores = {}
        for c in range(n_ch):
            if c >= NBUF - 1:
                for h in stores.pop(c - (NBUF - 1)):
                    h.wait()
                if c + 1 < n_ch:
                    loads[c + 1] = start_loads(c + 1)
            for h in loads.pop(c):
                h.wait()
            compute(c)
            stores[c] = start_stores(c)
        for hs in stores.values():
            for h in hs:
                h.wait()

    return k


def kernel(x, pe_weight):
    B, S, D = x.shape
    return _make_sc_kernel(B, S, D)(x, pe_weight[:S])


# speedup vs baseline: 1.3157x; 1.0625x over previous
import functools

import jax
import jax.numpy as jnp
from jax import lax
from jax.experimental import pallas as pl
from jax.experimental.pallas import tpu as pltpu
from jax.experimental.pallas import tpu_sc as plsc

LANES = 16
NBUF = 3
CH = 8


def _make_sc_kernel(B, S, D):
    info = plsc.get_sparse_core_info()
    NC, NS = info.num_cores, info.num_subcores
    NW = NC * NS
    s_per_w = S // NW
    n_ch = s_per_w // CH
    n_col = D // LANES

    mesh = plsc.VectorSubcoreMesh(core_axis_name="c", subcore_axis_name="s")

    scratch = (
        [pltpu.VMEM((B, CH, D), jnp.float32) for _ in range(NBUF)]
        + [pltpu.VMEM((CH, D), jnp.float32) for _ in range(NBUF)]
        + [pltpu.SemaphoreType.DMA for _ in range(2 * NBUF)]
    )

    @functools.partial(
        pl.kernel,
        mesh=mesh,
        out_type=jax.ShapeDtypeStruct((B, S, D), jnp.float32),
        scratch_types=scratch,
        compiler_params=pltpu.CompilerParams(use_tc_tiling_on_sc=True),
    )
    def k(xf, pe, out, *refs):
        xbs = refs[:NBUF]
        pbs = refs[NBUF:2 * NBUF]
        lss = refs[2 * NBUF:3 * NBUF]
        sss = refs[3 * NBUF:4 * NBUF]

        wid = lax.axis_index("s") * NC + lax.axis_index("c")
        s_base = wid * s_per_w

        def start_loads(c):
            p = c % NBUF
            s0 = s_base + c * CH
            return [
                pltpu.async_copy(pe.at[pl.ds(s0, CH), :], pbs[p], lss[p]),
                pltpu.async_copy(xf.at[:, pl.ds(s0, CH), :], xbs[p], lss[p]),
            ]

        def start_stores(c):
            p = c % NBUF
            s0 = s_base + c * CH
            return [
                pltpu.async_copy(xbs[p], out.at[:, pl.ds(s0, CH), :], sss[p]),
            ]

        GPB = 16

        def compute(c):
            p = c % NBUF
            xb, pb = xbs[p], pbs[p]

            def body(r, carry):
                def cbody(j, carry2):
                    base = j * (GPB * LANES)
                    for g in range(GPB):
                        col = base + g * LANES
                        vec = pb[r, pl.ds(col, LANES)]
                        for b in range(B):
                            plsc.addupdate(xb.at[b, r, pl.ds(col, LANES)], vec)
                    return carry2

                lax.fori_loop(0, n_col // GPB, cbody, 0)
                return carry

            lax.fori_loop(0, CH, body, 0)

        loads = {c: start_loads(c) for c in range(min(NBUF, n_ch))}
        stores = {}
        for c in range(n_ch):
            if c >= NBUF - 1:
                for h in stores.pop(c - (NBUF - 1)):
                    h.wait()
                if c + 1 < n_ch:
                    loads[c + 1] = start_loads(c + 1)
            for h in loads.pop(c):
                h.wait()
            compute(c)
            stores[c] = start_stores(c)
        for hs in stores.values():
            for h in hs:
                h.wait()

    return k


def kernel(x, pe_weight):
    B, S, D = x.shape
    return _make_sc_kernel(B, S, D)(x, pe_weight[:S])


# speedup vs baseline: 1.3512x; 1.0270x over previous
import functools

import jax
import jax.numpy as jnp
from jax import lax
from jax.experimental import pallas as pl
from jax.experimental.pallas import tpu as pltpu
from jax.experimental.pallas import tpu_sc as plsc

LANES = 16
NBUF = 3
CH = 8


def _make_sc_kernel(B, S, D):
    info = plsc.get_sparse_core_info()
    NC, NS = info.num_cores, info.num_subcores
    NW = NC * NS
    s_per_w = S // NW
    n_ch = s_per_w // CH
    n_col = D // LANES

    mesh = plsc.VectorSubcoreMesh(core_axis_name="c", subcore_axis_name="s")

    scratch = (
        [pltpu.VMEM((B, CH, D), jnp.float32) for _ in range(NBUF)]
        + [pltpu.VMEM((CH, D), jnp.float32) for _ in range(NBUF)]
        + [pltpu.SemaphoreType.DMA for _ in range(2 * NBUF)]
    )

    @functools.partial(
        pl.kernel,
        mesh=mesh,
        out_type=jax.ShapeDtypeStruct((B, S, D), jnp.float32),
        scratch_types=scratch,
        compiler_params=pltpu.CompilerParams(use_tc_tiling_on_sc=True),
    )
    def k(xf, pe, out, *refs):
        xbs = refs[:NBUF]
        pbs = refs[NBUF:2 * NBUF]
        lss = refs[2 * NBUF:3 * NBUF]
        sss = refs[3 * NBUF:4 * NBUF]

        wid = lax.axis_index("s") * NC + lax.axis_index("c")
        s_base = wid * s_per_w

        def start_loads(c):
            p = c % NBUF
            s0 = s_base + c * CH
            return [
                pltpu.async_copy(pe.at[pl.ds(s0, CH), :], pbs[p], lss[p]),
                pltpu.async_copy(xf.at[:, pl.ds(s0, CH), :], xbs[p], lss[p]),
            ]

        def start_stores(c):
            p = c % NBUF
            s0 = s_base + c * CH
            return [
                pltpu.async_copy(xbs[p], out.at[:, pl.ds(s0, CH), :], sss[p]),
            ]

        GPB = 8

        def compute(c):
            p = c % NBUF
            xb, pb = xbs[p], pbs[p]

            def body(r, carry):
                def cbody(j, carry2):
                    base = j * (GPB * LANES)
                    for g in range(GPB):
                        col = base + g * LANES
                        vec = pb[r, pl.ds(col, LANES)]
                        for b in range(B):
                            plsc.addupdate(xb.at[b, r, pl.ds(col, LANES)], vec)
                    return carry2

                lax.fori_loop(0, n_col // GPB, cbody, 0)
                return carry

            lax.fori_loop(0, CH, body, 0)

        loads = {c: start_loads(c) for c in range(min(NBUF, n_ch))}
        stores = {}
        for c in range(n_ch):
            if c >= NBUF - 1:
                for h in stores.pop(c - (NBUF - 1)):
                    h.wait()
                if c + 1 < n_ch:
                    loads[c + 1] = start_loads(c + 1)
            for h in loads.pop(c):
                h.wait()
            compute(c)
            stores[c] = start_stores(c)
        for hs in stores.values():
            for h in hs:
                h.wait()

    return k


def kernel(x, pe_weight):
    B, S, D = x.shape
    return _make_sc_kernel(B, S, D)(x, pe_weight[:S])


# speedup vs baseline: 1.3782x; 1.0200x over previous
import functools

import jax
import jax.numpy as jnp
from jax import lax
from jax.experimental import pallas as pl
from jax.experimental.pallas import tpu as pltpu
from jax.experimental.pallas import tpu_sc as plsc

LANES = 16
NBUF = 3
CH = 8


def _make_sc_kernel(B, S, D):
    info = plsc.get_sparse_core_info()
    NC, NS = info.num_cores, info.num_subcores
    NW = NC * NS
    s_per_w = S // NW
    n_ch = s_per_w // CH
    n_col = D // LANES

    mesh = plsc.VectorSubcoreMesh(core_axis_name="c", subcore_axis_name="s")

    scratch = (
        [pltpu.VMEM((B, CH, D), jnp.float32) for _ in range(NBUF)]
        + [pltpu.VMEM((CH, D), jnp.float32) for _ in range(NBUF)]
        + [pltpu.SemaphoreType.DMA for _ in range(2 * NBUF)]
    )

    @functools.partial(
        pl.kernel,
        mesh=mesh,
        out_type=jax.ShapeDtypeStruct((B, S, D), jnp.float32),
        scratch_types=scratch,
        compiler_params=pltpu.CompilerParams(use_tc_tiling_on_sc=True),
    )
    def k(xf, pe, out, *refs):
        xbs = refs[:NBUF]
        pbs = refs[NBUF:2 * NBUF]
        lss = refs[2 * NBUF:3 * NBUF]
        sss = refs[3 * NBUF:4 * NBUF]

        wid = lax.axis_index("s") * NC + lax.axis_index("c")
        s_base = wid * s_per_w

        def start_loads(c):
            p = c % NBUF
            s0 = s_base + c * CH
            return [
                pltpu.async_copy(pe.at[pl.ds(s0, CH), :], pbs[p], lss[p]),
                pltpu.async_copy(xf.at[:, pl.ds(s0, CH), :], xbs[p], lss[p]),
            ]

        def start_stores(c):
            p = c % NBUF
            s0 = s_base + c * CH
            return [
                pltpu.async_copy(xbs[p], out.at[:, pl.ds(s0, CH), :], sss[p]),
            ]

        GPB = 4

        def compute(c):
            p = c % NBUF
            xb, pb = xbs[p], pbs[p]

            def body(r, carry):
                def cbody(j, carry2):
                    base = j * (GPB * LANES)
                    for g in range(GPB):
                        col = base + g * LANES
                        vec = pb[r, pl.ds(col, LANES)]
                        for b in range(B):
                            plsc.addupdate(xb.at[b, r, pl.ds(col, LANES)], vec)
                    return carry2

                lax.fori_loop(0, n_col // GPB, cbody, 0)
                return carry

            lax.fori_loop(0, CH, body, 0)

        loads = {c: start_loads(c) for c in range(min(NBUF, n_ch))}
        stores = {}
        for c in range(n_ch):
            if c >= NBUF - 1:
                for h in stores.pop(c - (NBUF - 1)):
                    h.wait()
                if c + 1 < n_ch:
                    loads[c + 1] = start_loads(c + 1)
            for h in loads.pop(c):
                h.wait()
            compute(c)
            stores[c] = start_stores(c)
        for hs in stores.values():
            for h in hs:
                h.wait()

    return k


def kernel(x, pe_weight):
    B, S, D = x.shape
    return _make_sc_kernel(B, S, D)(x, pe_weight[:S])


# speedup vs baseline: 1.3784x; 1.0002x over previous
import functools

import jax
import jax.numpy as jnp
from jax import lax
from jax.experimental import pallas as pl
from jax.experimental.pallas import tpu as pltpu
from jax.experimental.pallas import tpu_sc as plsc

LANES = 16
NBUF = 3
CH = 8


def _make_sc_kernel(B, S, D):
    info = plsc.get_sparse_core_info()
    NC, NS = info.num_cores, info.num_subcores
    NW = NC * NS
    s_per_w = S // NW
    n_ch = s_per_w // CH
    n_col = D // LANES

    mesh = plsc.VectorSubcoreMesh(core_axis_name="c", subcore_axis_name="s")

    scratch = (
        [pltpu.VMEM((B, CH, D), jnp.float32) for _ in range(NBUF)]
        + [pltpu.VMEM((CH, D), jnp.float32) for _ in range(NBUF)]
        + [pltpu.SemaphoreType.DMA for _ in range(2 * NBUF)]
    )

    @functools.partial(
        pl.kernel,
        mesh=mesh,
        out_type=jax.ShapeDtypeStruct((B, S, D), jnp.float32),
        scratch_types=scratch,
        compiler_params=pltpu.CompilerParams(use_tc_tiling_on_sc=True),
    )
    def k(xf, pe, out, *refs):
        xbs = refs[:NBUF]
        pbs = refs[NBUF:2 * NBUF]
        lss = refs[2 * NBUF:3 * NBUF]
        sss = refs[3 * NBUF:4 * NBUF]

        wid = lax.axis_index("s") * NC + lax.axis_index("c")
        s_base = wid * s_per_w

        def start_loads(c):
            p = c % NBUF
            s0 = s_base + c * CH
            return [
                pltpu.async_copy(pe.at[pl.ds(s0, CH), :], pbs[p], lss[p]),
                pltpu.async_copy(xf.at[:, pl.ds(s0, CH), :], xbs[p], lss[p]),
            ]

        def start_stores(c):
            p = c % NBUF
            s0 = s_base + c * CH
            return [
                pltpu.async_copy(xbs[p], out.at[:, pl.ds(s0, CH), :], sss[p]),
            ]

        GPB = 2

        def compute(c):
            p = c % NBUF
            xb, pb = xbs[p], pbs[p]

            def body(r, carry):
                def cbody(j, carry2):
                    base = j * (GPB * LANES)
                    for g in range(GPB):
                        col = base + g * LANES
                        vec = pb[r, pl.ds(col, LANES)]
                        for b in range(B):
                            plsc.addupdate(xb.at[b, r, pl.ds(col, LANES)], vec)
                    return carry2

                lax.fori_loop(0, n_col // GPB, cbody, 0)
                return carry

            lax.fori_loop(0, CH, body, 0)

        loads = {c: start_loads(c) for c in range(min(NBUF, n_ch))}
        stores = {}
        for c in range(n_ch):
            if c >= NBUF - 1:
                for h in stores.pop(c - (NBUF - 1)):
                    h.wait()
                if c + 1 < n_ch:
                    loads[c + 1] = start_loads(c + 1)
            for h in loads.pop(c):
                h.wait()
            compute(c)
            stores[c] = start_stores(c)
        for hs in stores.values():
            for h in hs:
                h.wait()

    return k


def kernel(x, pe_weight):
    B, S, D = x.shape
    return _make_sc_kernel(B, S, D)(x, pe_weight[:S])
